# Initial kernel scaffold; baseline (speedup 1.0000x reference)
import jax
import jax.numpy as jnp
from jax.experimental import pallas as pl


def kernel(q, mu, pairlist, W_in, b_in, W_e1, b_e1, W_e2, b_e2, W_att, b_att, W_mix, W_pn1, b_pn1, W_pn2, b_pn2, W_n1, b_n1, W_n2, b_n2):
    raise NotImplementedError("write your pallas kernel here")


# speedup vs baseline: 4.8970x; 4.8970x over previous
import functools

import jax
import jax.numpy as jnp
from jax import lax
from jax.experimental import pallas as pl
from jax.experimental.pallas import tpu as pltpu
from jax.experimental.pallas import tpu_sc as plsc

N = 10000
E = 160000
D = 128
H = 2
C = H * D
NRBF = 20
CUTOFF = 5.0

NC = 2
NS = 16
NW = NC * NS

E_PAD = 163840
EPW = E_PAD // NW
CH = 1024
NB = 320
JUNK = NB
ACCR = 328
CHP = 2048

BE = 2048
BN = 1000

@functools.lru_cache(maxsize=None)
def _get_mesh():
    return plsc.VectorSubcoreMesh(core_axis_name="c", subcore_axis_name="s",
                                  num_cores=NC, num_subcores=NS)


def _silu(x):
    return x * jax.nn.sigmoid(x)


@functools.lru_cache(maxsize=None)
def _make_gather(width):
    @functools.partial(
        pl.kernel,
        mesh=_get_mesh(),
        out_type=jax.ShapeDtypeStruct((E_PAD, width), jnp.float32),
        scratch_types=[
            pltpu.VMEM((CH // 128, 128), jnp.int32),
            pltpu.VMEM((128, width), jnp.float32),
            pltpu.SemaphoreType.DMA,
        ],
    )
    def gather_k(table_hbm, idx2_hbm, out_hbm, idx_v, rows_v, sem):
        wid = lax.axis_index("s") * NC + lax.axis_index("c")
        rbase = wid * (EPW // 128)
        nrow = CH // 128

        def body(k, carry):
            r0 = rbase + k * nrow
            pltpu.sync_copy(idx2_hbm.at[pl.ds(r0, nrow)], idx_v)
            for r in range(nrow):
                pltpu.async_copy(table_hbm.at[idx_v.at[r]], rows_v, sem).wait()
                pltpu.sync_copy(rows_v, out_hbm.at[pl.ds((r0 + r) * 128, 128)])
            return carry

        lax.fori_loop(0, EPW // CH, body, 0)

    return gather_k


@functools.lru_cache(maxsize=None)
def _make_scatter_wide(width):
    nburst = width // 16

    @functools.partial(
        pl.kernel,
        mesh=_get_mesh(),
        compiler_params=pltpu.CompilerParams(needs_layout_passes=False),
        out_type=jax.ShapeDtypeStruct((N, width), jnp.float32),
        scratch_types=[
            pltpu.VMEM((CHP,), jnp.int32),
            pltpu.VMEM((CHP + 512,), jnp.int32),
            pltpu.VMEM((128,), jnp.int32),
            pltpu.VMEM((128, width), jnp.float32),
            pltpu.VMEM((ACCR, width), jnp.float32),
            pltpu.SemaphoreType.DMA,
        ],
    )
    def scatter_k(jflat_hbm, pay_hbm, out_hbm, idx_v, pk_v, eid_v, pay_v,
                  acc, sem):
        w = lax.axis_index("c") * NS + lax.axis_index("s")
        lo = w * NB
        lane = lax.iota(jnp.int32, 16)

        def zb(i, carry):
            for cc in range(nburst):
                acc[i, pl.ds(cc * 16, 16)] = jnp.zeros((16,), jnp.float32)
            return carry

        lax.fori_loop(0, ACCR, zb, 0)

        def drain(off):
            def batch(b, carry):
                def eidg(g, c2):
                    v = pk_v[pl.ds(b * 128 + g * 16, 16)]
                    eid_v[pl.ds(g * 16, 16)] = (
                        jax.lax.shift_right_logical(v, 9))
                    return c2

                lax.fori_loop(0, 8, eidg, 0)
                pltpu.async_copy(pay_hbm.at[eid_v], pay_v, sem).wait()

                def accg(g, c2):
                    v = pk_v[pl.ds(b * 128 + g * 16, 16)]
                    jlv = v & 511
                    for l in range(16):
                        jl = jlv[l]
                        prow = g * 16 + l
                        for cc in range(nburst):
                            acc[jl, pl.ds(cc * 16, 16)] = (
                                acc[jl, pl.ds(cc * 16, 16)]
                                + pay_v[prow, pl.ds(cc * 16, 16)])
                    return c2

                lax.fori_loop(0, 8, accg, 0)
                return carry

            nb2 = off // 128
            lax.fori_loop(0, nb2, batch, 0)
            rem = off - nb2 * 128

            def mv(kk, carry):
                pk_v[pl.ds(kk * 16, 16)] = pk_v[pl.ds(nb2 * 128 + kk * 16, 16)]
                return carry

            lax.fori_loop(0, (rem + 15) // 16, mv, 0)
            return rem

        def chunk(kk, off):
            e0 = kk * CHP
            pltpu.sync_copy(jflat_hbm.at[pl.ds(e0, CHP)], idx_v)

            def grp(g, off2):
                v = idx_v[pl.ds(g * 16, 16)]
                pos = e0 + g * 16 + lane
                ok = (v >= lo) & (v < lo + NB) & (pos < E)
                pk = jax.lax.shift_left(pos, 9) | jnp.where(ok, v - lo, JUNK)
                ps = plsc.cumsum(jnp.where(ok, 1, 0))
                plsc.store_scatter(pk_v, [off2 + ps - 1], pk, mask=ok)
                cntv = plsc.all_reduce_population_count(ok)
                return off2 + cntv[0]

            off = lax.fori_loop(0, CHP // 16, grp, off)
            return drain(off)

        off = lax.fori_loop(0, E_PAD // CHP, chunk, 0)

        def padg(g, carry):
            pos = E + ((w * 128 + g * 16) % (E_PAD - E)) + lane
            pk = jax.lax.shift_left(pos, 9) | JUNK
            pk_v[pl.ds(off + g * 16, 16)] = pk
            return carry

        lax.fori_loop(0, 8, padg, 0)
        drain(off + 128)

        @pl.when(w < NW - 1)
        def _():
            pltpu.sync_copy(acc.at[pl.ds(0, NB)], out_hbm.at[pl.ds(lo, NB)])

        @pl.when(w == NW - 1)
        def _():
            pltpu.sync_copy(acc.at[pl.ds(0, N - NB * (NW - 1))],
                            out_hbm.at[pl.ds(lo, N - NB * (NW - 1))])

    return scatter_k


def _make_scatter_small():
    @functools.partial(
        pl.kernel,
        mesh=_get_mesh(),
        compiler_params=pltpu.CompilerParams(needs_layout_passes=False),
        out_type=jax.ShapeDtypeStruct((N * 16,), jnp.float32),
        scratch_types=[
            pltpu.VMEM((CHP,), jnp.int32),
            pltpu.VMEM((CHP + 16,), jnp.int32),
            pltpu.VMEM((CHP * 16,), jnp.float32),
            pltpu.VMEM((ACCR * 16,), jnp.float32),
            pltpu.SemaphoreType.DMA,
        ],
    )
    def scatter_k(jflat_hbm, pay_hbm, out_hbm, idx_v, pk_v, pay_v, acc, sem):
        w = lax.axis_index("c") * NS + lax.axis_index("s")
        lo = w * NB
        lane = lax.iota(jnp.int32, 16)

        def zb(i, carry):
            acc[pl.ds(i * 16, 16)] = jnp.zeros((16,), jnp.float32)
            return carry

        lax.fori_loop(0, ACCR, zb, 0)

        def chunk(kk, carry):
            e0 = kk * CHP
            pltpu.sync_copy(jflat_hbm.at[pl.ds(e0, CHP)], idx_v)
            pltpu.sync_copy(pay_hbm.at[pl.ds(e0 * 16, CHP * 16)], pay_v)

            def grp(g, off2):
                v = idx_v[pl.ds(g * 16, 16)]
                pos = e0 + g * 16 + lane
                lid = g * 16 + lane
                ok = (v >= lo) & (v < lo + NB) & (pos < E)
                pk = jax.lax.shift_left(lid, 9) | jnp.where(ok, v - lo, JUNK)
                ps = plsc.cumsum(jnp.where(ok, 1, 0))
                plsc.store_scatter(pk_v, [off2 + ps - 1], pk, mask=ok)
                cntv = plsc.all_reduce_population_count(ok)
                return off2 + cntv[0]

            cnt = lax.fori_loop(0, CHP // 16, grp, 0)
            pk_v[pl.ds(cnt, 16)] = jnp.full((16,), JUNK, jnp.int32)

            def accg(g, c2):
                v = pk_v[pl.ds(g * 16, 16)]
                jlv = v & 511
                lidv = jax.lax.shift_right_logical(v, 9)
                for l in range(16):
                    jl = jlv[l]
                    lid = lidv[l]
                    acc[pl.ds(jl * 16, 16)] = (
                        acc[pl.ds(jl * 16, 16)]
                        + pay_v[pl.ds(lid * 16, 16)])
                return c2

            lax.fori_loop(0, (cnt + 15) // 16, accg, 0)
            return carry

        lax.fori_loop(0, E_PAD // CHP, chunk, 0)

        @pl.when(w < NW - 1)
        def _():
            pltpu.sync_copy(acc.at[pl.ds(0, NB * 16)],
                            out_hbm.at[pl.ds(lo * 16, NB * 16)])

        @pl.when(w == NW - 1)
        def _():
            pltpu.sync_copy(acc.at[pl.ds(0, (N - NB * (NW - 1)) * 16)],
                            out_hbm.at[pl.ds(lo * 16, (N - NB * (NW - 1)) * 16)])

    return scatter_k


_scatter_small_cache = []


def _get_scatter_small():
    if not _scatter_small_cache:
        _scatter_small_cache.append(_make_scatter_small())
    return _scatter_small_cache[0]


_OFF2 = (CUTOFF / (NRBF - 1)) ** 2


def edge1_body(qi_ref, qj_ref, wina_ref, winb_ref, bin_ref, cent_ref,
               wea_ref, web_ref, wec_ref, wed_ref, be1_ref, we2_ref, be2_ref,
               watt_ref, batt_ref, mtx_ref, small_ref):
    qi = qi_ref[...]
    qj = qj_ref[...]
    r = qj - qi
    d = jnp.sqrt(jnp.sum(r * r, axis=1, keepdims=True))
    qlin = (jnp.dot(qi, wina_ref[...], preferred_element_type=jnp.float32)
            + jnp.dot(qj, winb_ref[...], preferred_element_type=jnp.float32)
            + bin_ref[...])
    diff = d - cent_ref[...]
    rbf = jnp.exp((-0.5 / _OFF2) * diff * diff)
    qfilt = rbf * qlin
    pre = (jnp.dot(qi, wea_ref[...], preferred_element_type=jnp.float32)
           + jnp.dot(qj, web_ref[...], preferred_element_type=jnp.float32)
           + jnp.dot(qfilt, wec_ref[...], preferred_element_type=jnp.float32)
           + d * wed_ref[...] + be1_ref[...])
    mtx = (jnp.dot(_silu(pre), we2_ref[...], preferred_element_type=jnp.float32)
           + be2_ref[...])
    z = (jnp.dot(mtx, watt_ref[...], preferred_element_type=jnp.float32)
         + batt_ref[...])
    att = jnp.where(z > 0, z, 2.0 * (jnp.exp(0.5 * z) - 1.0))
    ea = jnp.exp(att)
    rn = r[:, :3] / (d + 1e-5)
    nb = qi.shape[0]
    small = jnp.concatenate(
        [ea, rn, jnp.ones((nb, 1), jnp.float32),
         jnp.zeros((nb, 10), jnp.float32)], axis=1)
    mtx_ref[...] = mtx
    small_ref[...] = small


def edge2_body(mtx_ref, small_ref, sg_ref, wm00_ref, wm01_ref, wm10_ref,
               wm11_ref, p1_ref, p2_ref, p3_ref, p4_ref):
    mtx = mtx_ref[...]
    small = small_ref[...]
    sg = sg_ref[...]
    c0 = small[:, 0:1] * sg[:, 0:1]
    c1 = small[:, 1:2] * sg[:, 1:2]
    m0 = mtx * c0
    m1 = mtx * c1
    pre0 = (jnp.dot(m0, wm00_ref[...], preferred_element_type=jnp.float32)
            + jnp.dot(m1, wm10_ref[...], preferred_element_type=jnp.float32))
    pre1 = (jnp.dot(m0, wm01_ref[...], preferred_element_type=jnp.float32)
            + jnp.dot(m1, wm11_ref[...], preferred_element_type=jnp.float32))
    co0 = jnp.tanh(pre0)
    co1 = jnp.tanh(pre1)
    rn0 = small[:, 2:3]
    rn1 = small[:, 3:4]
    rn2 = small[:, 4:5]
    nb = mtx.shape[0]
    pos = pl.program_id(0) * nb + jax.lax.broadcasted_iota(
        jnp.int32, (nb, 1), 0)
    live = pos < E
    p1_ref[...] = jnp.where(live, jnp.concatenate([m0, m1], axis=1), 0.0)
    p2_ref[...] = jnp.where(
        live, jnp.concatenate([co0 * rn0, co0 * rn1], axis=1), 0.0)
    p3_ref[...] = jnp.where(
        live, jnp.concatenate([co0 * rn2, co1 * rn0], axis=1), 0.0)
    p4_ref[...] = jnp.where(
        live, jnp.concatenate([co1 * rn1, co1 * rn2], axis=1), 0.0)


def sinv_body(s_ref, out_ref):
    s = s_ref[...]
    nb = s.shape[0]
    out_ref[...] = jnp.concatenate(
        [1.0 / s[:, 0:2], jnp.zeros((nb, D - 2), jnp.float32)], axis=1)


def node_body(q_ref, am_ref, a2_ref, a3_ref, a4_ref, s_ref,
              p0_ref, p1_ref, bpn1_ref, wpn2_ref, bpn2_ref,
              wa_ref, g0_ref, g1_ref, cc_ref, bn1_ref, wn2_ref, bn2_ref,
              out_ref):
    q = q_ref[...]
    am = am_ref[...]
    a2 = a2_ref[...]
    a3 = a3_ref[...]
    a4 = a4_ref[...]
    inv = 1.0 / jnp.maximum(s_ref[...][:, 5:6], 1.0)
    u00 = a2[:, :D] * inv
    u01 = a2[:, D:] * inv
    u02 = a3[:, :D] * inv
    u10 = a3[:, D:] * inv
    u11 = a4[:, :D] * inv
    u12 = a4[:, D:] * inv
    cn0 = u00 * u00 + u01 * u01 + u02 * u02
    cn1 = u10 * u10 + u11 * u11 + u12 * u12
    t = _silu(jnp.dot(cn0, p0_ref[...], preferred_element_type=jnp.float32)
              + jnp.dot(cn1, p1_ref[...], preferred_element_type=jnp.float32)
              + bpn1_ref[...])
    q_comb = _silu(jnp.dot(t, wpn2_ref[...], preferred_element_type=jnp.float32)
                   + bpn2_ref[...])
    h = _silu(jnp.dot(q, wa_ref[...], preferred_element_type=jnp.float32)
              + jnp.dot(am[:, :D], g0_ref[...], preferred_element_type=jnp.float32)
              + jnp.dot(am[:, D:], g1_ref[...], preferred_element_type=jnp.float32)
              + jnp.dot(q_comb, cc_ref[...], preferred_element_type=jnp.float32)
              + bn1_ref[...])
    out = _silu(jnp.dot(h, wn2_ref[...], preferred_element_type=jnp.float32)
                + bn2_ref[...])
    out_ref[...] = 2.0 * q + out


def _full(shape):
    return pl.BlockSpec(shape, lambda i: tuple(0 for _ in shape))


def _blk(shape):
    return pl.BlockSpec(shape, lambda i: (i,) + tuple(0 for _ in shape[1:]))


def _tc_edge1(qi, qj, wina, winb, binr, cent, wea, web, wec, wed, be1, we2,
              be2, watt, batt):
    grid = (E_PAD // BE,)
    return pl.pallas_call(
        edge1_body,
        grid=grid,
        in_specs=[
            _blk((BE, D)), _blk((BE, D)),
            _full((D, NRBF)), _full((D, NRBF)), _full((1, NRBF)),
            _full((1, NRBF)),
            _full((D, D)), _full((D, D)), _full((NRBF, D)), _full((1, D)),
            _full((1, D)), _full((D, D)), _full((1, D)),
            _full((D, H)), _full((1, H)),
        ],
        out_specs=[_blk((BE, D)), _blk((BE, 16))],
        out_shape=[
            jax.ShapeDtypeStruct((E_PAD, D), jnp.float32),
            jax.ShapeDtypeStruct((E_PAD, 16), jnp.float32),
        ],
    )(qi, qj, wina, winb, binr, cent, wea, web, wec, wed, be1, we2, be2,
      watt, batt)


def _tc_sinv(s):
    return pl.pallas_call(
        sinv_body,
        grid=(N // BN,),
        in_specs=[_blk((BN, 16))],
        out_specs=_blk((BN, D)),
        out_shape=jax.ShapeDtypeStruct((N, D), jnp.float32),
    )(s)


def _tc_edge2(mtx, small, sg, wm00, wm01, wm10, wm11):
    grid = (E_PAD // BE,)
    return pl.pallas_call(
        edge2_body,
        grid=grid,
        in_specs=[
            _blk((BE, D)), _blk((BE, 16)), _blk((BE, D)),
            _full((D, D)), _full((D, D)), _full((D, D)), _full((D, D)),
        ],
        out_specs=[_blk((BE, C)), _blk((BE, C)), _blk((BE, C)), _blk((BE, C))],
        out_shape=[jax.ShapeDtypeStruct((E_PAD, C), jnp.float32)] * 4,
    )(mtx, small, sg, wm00, wm01, wm10, wm11)


def _tc_node(q, am, a2, a3, a4, s, p0, p1, bpn1, wpn2, bpn2, wa, g0, g1, cc,
             bn1, wn2, bn2):
    grid = (N // BN,)
    return pl.pallas_call(
        node_body,
        grid=grid,
        in_specs=[
            _blk((BN, D)), _blk((BN, C)), _blk((BN, C)), _blk((BN, C)),
            _blk((BN, C)), _blk((BN, 16)),
            _full((D, D)), _full((D, D)), _full((1, D)), _full((D, D)),
            _full((1, D)),
            _full((D, D)), _full((D, D)), _full((D, D)), _full((D, D)),
            _full((1, D)), _full((D, D)), _full((1, D)),
        ],
        out_specs=_blk((BN, D)),
        out_shape=jax.ShapeDtypeStruct((N, D), jnp.float32),
    )(q, am, a2, a3, a4, s, p0, p1, bpn1, wpn2, bpn2, wa, g0, g1, cc, bn1,
      wn2, bn2)


def _gather_q(table, idx2):
    return _make_gather(D)(table, idx2)


def _gather_s(table, idx2):
    return _make_gather(D)(table, idx2)


def _scatter16(jflat, pay):
    return _get_scatter_small()(jflat, pay.reshape(-1)).reshape(N, 16)


def _scatter256(jflat, pay):
    return _make_scatter_wide(C)(jflat, pay)


def kernel(q, mu, pairlist, W_in, b_in, W_e1, b_e1, W_e2, b_e2, W_att, b_att,
           W_mix, W_pn1, b_pn1, W_pn2, b_pn2, W_n1, b_n1, W_n2, b_n2):
    idx_i = pairlist[0].astype(jnp.int32)
    idx_j = pairlist[1].astype(jnp.int32)
    padz = jnp.zeros((E_PAD - E,), jnp.int32)
    iflat = jnp.concatenate([idx_i, padz])
    jflat = jnp.concatenate([idx_j, padz])
    ii2 = iflat.reshape(E_PAD // 128, 128)
    jj2 = jflat.reshape(E_PAD // 128, 128)

    wina, winb = W_in[:D], W_in[D:]
    binr = b_in.reshape(1, NRBF)
    cent = jnp.linspace(0.0, CUTOFF, NRBF).reshape(1, NRBF)
    wea, web = W_e1[:D], W_e1[D:2 * D]
    wec = W_e1[2 * D:2 * D + NRBF]
    wed = W_e1[2 * D + NRBF].reshape(1, D)
    be1 = b_e1.reshape(1, D)
    be2 = b_e2.reshape(1, D)
    batt = b_att.reshape(1, H)
    wm00 = W_mix[0::2, 0::2]
    wm01 = W_mix[0::2, 1::2]
    wm10 = W_mix[1::2, 0::2]
    wm11 = W_mix[1::2, 1::2]
    p0, p1 = W_pn1[0::2], W_pn1[1::2]
    bpn1 = b_pn1.reshape(1, D)
    bpn2 = b_pn2.reshape(1, D)
    wa = W_n1[:D]
    g0 = W_n1[D:D + C][0::2]
    g1 = W_n1[D:D + C][1::2]
    cc = W_n1[D + C:]
    bn1 = b_n1.reshape(1, D)
    bn2 = b_n2.reshape(1, D)

    qi = _gather_q(q, ii2)
    qj = _gather_q(q, jj2)
    mtx, small = _tc_edge1(qi, qj, wina, winb, binr, cent, wea, web, wec,
                           wed, be1, W_e2, be2, W_att, batt)
    s_sum = _scatter16(jflat, small)
    sg = _gather_s(_tc_sinv(s_sum), jj2)
    pay1, pay2, pay3, pay4 = _tc_edge2(mtx, small, sg, wm00, wm01, wm10, wm11)
    am = _scatter256(jflat, pay1)
    a2 = _scatter256(jflat, pay2)
    a3 = _scatter256(jflat, pay3)
    a4 = _scatter256(jflat, pay4)
    q_new = _tc_node(q, am, a2, a3, a4, s_sum, p0, p1, bpn1, W_pn2, bpn2,
                     wa, g0, g1, cc, bn1, W_n2, bn2)
    return (q_new, mu)


# speedup vs baseline: 6.4815x; 1.3236x over previous
import functools

import jax
import jax.numpy as jnp
from jax import lax
from jax.experimental import pallas as pl
from jax.experimental.pallas import tpu as pltpu
from jax.experimental.pallas import tpu_sc as plsc

N = 10000
E = 160000
D = 128
H = 2
C = H * D
NRBF = 20
CUTOFF = 5.0

NC = 2
NS = 16
NW = NC * NS

E_PAD = 163840
EPW = E_PAD // NW
CH = 1024
NB = 320
JUNK = NB
ACCR = 328
CHP = 2048

BE = 2048
BN = 1000

@functools.lru_cache(maxsize=None)
def _get_mesh():
    return plsc.VectorSubcoreMesh(core_axis_name="c", subcore_axis_name="s",
                                  num_cores=NC, num_subcores=NS)


def _silu(x):
    return x * jax.nn.sigmoid(x)


@functools.lru_cache(maxsize=None)
def _make_gather(width):
    nbat = EPW // 128

    @functools.partial(
        pl.kernel,
        mesh=_get_mesh(),
        out_type=jax.ShapeDtypeStruct((E_PAD, width), jnp.float32),
        scratch_types=[
            pltpu.VMEM((nbat, 128), jnp.int32),
            pltpu.VMEM((128, width), jnp.float32),
            pltpu.VMEM((128, width), jnp.float32),
            pltpu.VMEM((128, width), jnp.float32),
            pltpu.VMEM((128, width), jnp.float32),
            pltpu.SemaphoreType.DMA,
            pltpu.SemaphoreType.DMA,
            pltpu.SemaphoreType.DMA,
            pltpu.SemaphoreType.DMA,
        ],
    )
    def gather_k(table_hbm, idx2_hbm, out_hbm, idx_v, b0, b1, b2, b3,
                 s0, s1, s2, s3):
        wid = lax.axis_index("s") * NC + lax.axis_index("c")
        rbase = wid * nbat
        bufs = (b0, b1, b2, b3)
        sems = (s0, s1, s2, s3)

        pltpu.sync_copy(idx2_hbm.at[pl.ds(rbase, nbat)], idx_v)
        for i in range(4):
            pltpu.async_copy(table_hbm.at[idx_v.at[i]], bufs[i], sems[i])

        def body(p, carry):
            for i in range(4):
                b = p * 4 + i
                pltpu.make_async_copy(table_hbm.at[idx_v.at[b]],
                                      bufs[i], sems[i]).wait()
                pltpu.sync_copy(bufs[i],
                                out_hbm.at[pl.ds((rbase + b) * 128, 128)])
                nxt = b + 4

                @pl.when(nxt < nbat)
                def _():
                    pltpu.async_copy(table_hbm.at[idx_v.at[nxt]],
                                     bufs[i], sems[i])
            return carry

        lax.fori_loop(0, nbat // 4, body, 0)

    return gather_k


CAP = 8192


def _make_compact():
    @functools.partial(
        pl.kernel,
        mesh=_get_mesh(),
        compiler_params=pltpu.CompilerParams(needs_layout_passes=False),
        out_type=[jax.ShapeDtypeStruct((NW * CAP,), jnp.int32),
                  jax.ShapeDtypeStruct((NW * 16,), jnp.int32)],
        scratch_types=[
            pltpu.VMEM((CHP,), jnp.int32),
            pltpu.VMEM((CAP,), jnp.int32),
            pltpu.VMEM((16,), jnp.int32),
            pltpu.SemaphoreType.DMA,
        ],
    )
    def compact_k(jflat_hbm, pk_hbm, cnt_hbm, idx_v, pk_v, cnt_v, sem):
        w = lax.axis_index("c") * NS + lax.axis_index("s")
        lo = w * NB
        lane = lax.iota(jnp.int32, 16)

        def chunk(kk, off):
            e0 = kk * CHP
            pltpu.sync_copy(jflat_hbm.at[pl.ds(e0, CHP)], idx_v)

            def grp(g, off2):
                v = idx_v[pl.ds(g * 16, 16)]
                pos = e0 + g * 16 + lane
                ok = (v >= lo) & (v < lo + NB) & (pos < E)
                pk = jax.lax.shift_left(pos, 9) | jnp.where(ok, v - lo, JUNK)
                ps = plsc.cumsum(jnp.where(ok, 1, 0))
                tgt = off2 + ps - 1
                ok = ok & (tgt < CAP - 128)
                plsc.store_scatter(pk_v, [tgt], pk, mask=ok)
                cntv = plsc.all_reduce_population_count(ok)
                return off2 + cntv[0]

            return lax.fori_loop(0, CHP // 16, grp, off)

        off = lax.fori_loop(0, E_PAD // CHP, chunk, 0)

        def padg(g, carry):
            pos = E + ((w * 128 + g * 16) % (E_PAD - E)) + lane
            pk = jax.lax.shift_left(pos, 9) | JUNK
            pk_v[pl.ds(off + g * 16, 16)] = pk
            return carry

        lax.fori_loop(0, 8, padg, 0)
        cntp = jnp.maximum(((off + 127) // 128) * 128, 128)
        cnt_v[pl.ds(0, 16)] = jnp.full((16,), 0, jnp.int32) + cntp
        pltpu.sync_copy(pk_v, pk_hbm.at[pl.ds(w * CAP, CAP)])
        pltpu.sync_copy(cnt_v, cnt_hbm.at[pl.ds(w * 16, 16)])

    return compact_k


@functools.lru_cache(maxsize=None)
def _make_scatter_wide(width):
    nburst = width // 16

    @functools.partial(
        pl.kernel,
        mesh=_get_mesh(),
        compiler_params=pltpu.CompilerParams(needs_layout_passes=False),
        out_type=jax.ShapeDtypeStruct((N, width), jnp.float32),
        scratch_types=[
            pltpu.VMEM((CAP,), jnp.int32),
            pltpu.VMEM((16,), jnp.int32),
            pltpu.VMEM((64,), jnp.int32),
            pltpu.VMEM((64,), jnp.int32),
            pltpu.VMEM((64, width), jnp.float32),
            pltpu.VMEM((64, width), jnp.float32),
            pltpu.VMEM((ACCR, width), jnp.float32),
            pltpu.SemaphoreType.DMA,
            pltpu.SemaphoreType.DMA,
        ],
    )
    def scatter_k(pk_hbm, cnt_hbm, pay_hbm, out_hbm, pk_v, cnt_v,
                  e0v, e1v, p0v, p1v, acc, sm0, sm1):
        w = lax.axis_index("c") * NS + lax.axis_index("s")
        lo = w * NB
        eids = (e0v, e1v)
        pays = (p0v, p1v)
        sems = (sm0, sm1)

        pltpu.sync_copy(pk_hbm.at[pl.ds(w * CAP, CAP)], pk_v)
        pltpu.sync_copy(cnt_hbm.at[pl.ds(w * 16, 16)], cnt_v)
        cnt = cnt_v[pl.ds(0, 16)][0]
        nbat = cnt // 64

        def zb(i, carry):
            for cc in range(nburst):
                acc[i, pl.ds(cc * 16, 16)] = jnp.zeros((16,), jnp.float32)
            return carry

        lax.fori_loop(0, ACCR, zb, 0)

        def fire(b, h):
            def eidg(g, c2):
                v = pk_v[pl.ds(b * 64 + g * 16, 16)]
                eids[h][pl.ds(g * 16, 16)] = jax.lax.shift_right_logical(v, 9)
                return c2

            lax.fori_loop(0, 4, eidg, 0)
            pltpu.async_copy(pay_hbm.at[eids[h]], pays[h], sems[h])

        fire(0, 0)
        fire(1, 1)

        def body(p, carry):
            for h in range(2):
                b = p * 2 + h
                pltpu.make_async_copy(pay_hbm.at[eids[h]], pays[h],
                                      sems[h]).wait()

                def accg(g, c2):
                    v = pk_v[pl.ds(b * 64 + g * 16, 16)]
                    jlv = v & 511
                    for l in range(16):
                        jl = jlv[l]
                        prow = g * 16 + l
                        for cc in range(nburst):
                            acc[jl, pl.ds(cc * 16, 16)] = (
                                acc[jl, pl.ds(cc * 16, 16)]
                                + pays[h][prow, pl.ds(cc * 16, 16)])
                    return c2

                lax.fori_loop(0, 4, accg, 0)
                nxt = b + 2

                @pl.when(nxt < nbat)
                def _():
                    fire(nxt, h)
            return carry

        lax.fori_loop(0, nbat // 2, body, 0)

        @pl.when(w < NW - 1)
        def _():
            pltpu.sync_copy(acc.at[pl.ds(0, NB)], out_hbm.at[pl.ds(lo, NB)])

        @pl.when(w == NW - 1)
        def _():
            pltpu.sync_copy(acc.at[pl.ds(0, N - NB * (NW - 1))],
                            out_hbm.at[pl.ds(lo, N - NB * (NW - 1))])

    return scatter_k


def _make_scatter_small():
    @functools.partial(
        pl.kernel,
        mesh=_get_mesh(),
        compiler_params=pltpu.CompilerParams(needs_layout_passes=False),
        out_type=jax.ShapeDtypeStruct((N * 16,), jnp.float32),
        scratch_types=[
            pltpu.VMEM((CHP,), jnp.int32),
            pltpu.VMEM((CHP + 16,), jnp.int32),
            pltpu.VMEM((CHP * 16,), jnp.float32),
            pltpu.VMEM((ACCR * 16,), jnp.float32),
            pltpu.SemaphoreType.DMA,
        ],
    )
    def scatter_k(jflat_hbm, pay_hbm, out_hbm, idx_v, pk_v, pay_v, acc, sem):
        w = lax.axis_index("c") * NS + lax.axis_index("s")
        lo = w * NB
        lane = lax.iota(jnp.int32, 16)

        def zb(i, carry):
            acc[pl.ds(i * 16, 16)] = jnp.zeros((16,), jnp.float32)
            return carry

        lax.fori_loop(0, ACCR, zb, 0)

        def chunk(kk, carry):
            e0 = kk * CHP
            pltpu.sync_copy(jflat_hbm.at[pl.ds(e0, CHP)], idx_v)
            pltpu.sync_copy(pay_hbm.at[pl.ds(e0 * 16, CHP * 16)], pay_v)

            def grp(g, off2):
                v = idx_v[pl.ds(g * 16, 16)]
                pos = e0 + g * 16 + lane
                lid = g * 16 + lane
                ok = (v >= lo) & (v < lo + NB) & (pos < E)
                pk = jax.lax.shift_left(lid, 9) | jnp.where(ok, v - lo, JUNK)
                ps = plsc.cumsum(jnp.where(ok, 1, 0))
                plsc.store_scatter(pk_v, [off2 + ps - 1], pk, mask=ok)
                cntv = plsc.all_reduce_population_count(ok)
                return off2 + cntv[0]

            cnt = lax.fori_loop(0, CHP // 16, grp, 0)
            pk_v[pl.ds(cnt, 16)] = jnp.full((16,), JUNK, jnp.int32)

            def accg(g, c2):
                v = pk_v[pl.ds(g * 16, 16)]
                jlv = v & 511
                lidv = jax.lax.shift_right_logical(v, 9)
                for l in range(16):
                    jl = jlv[l]
                    lid = lidv[l]
                    acc[pl.ds(jl * 16, 16)] = (
                        acc[pl.ds(jl * 16, 16)]
                        + pay_v[pl.ds(lid * 16, 16)])
                return c2

            lax.fori_loop(0, (cnt + 15) // 16, accg, 0)
            return carry

        lax.fori_loop(0, E_PAD // CHP, chunk, 0)

        @pl.when(w < NW - 1)
        def _():
            pltpu.sync_copy(acc.at[pl.ds(0, NB * 16)],
                            out_hbm.at[pl.ds(lo * 16, NB * 16)])

        @pl.when(w == NW - 1)
        def _():
            pltpu.sync_copy(acc.at[pl.ds(0, (N - NB * (NW - 1)) * 16)],
                            out_hbm.at[pl.ds(lo * 16, (N - NB * (NW - 1)) * 16)])

    return scatter_k


_scatter_small_cache = []


def _get_scatter_small():
    if not _scatter_small_cache:
        _scatter_small_cache.append(_make_scatter_small())
    return _scatter_small_cache[0]


_OFF2 = (CUTOFF / (NRBF - 1)) ** 2


def edge1_body(qi_ref, qj_ref, wina_ref, winb_ref, bin_ref, cent_ref,
               wea_ref, web_ref, wec_ref, wed_ref, be1_ref, we2_ref, be2_ref,
               watt_ref, batt_ref, mtx_ref, small_ref):
    qi = qi_ref[...]
    qj = qj_ref[...]
    r = qj - qi
    d = jnp.sqrt(jnp.sum(r * r, axis=1, keepdims=True))
    qlin = (jnp.dot(qi, wina_ref[...], preferred_element_type=jnp.float32)
            + jnp.dot(qj, winb_ref[...], preferred_element_type=jnp.float32)
            + bin_ref[...])
    diff = d - cent_ref[...]
    rbf = jnp.exp((-0.5 / _OFF2) * diff * diff)
    qfilt = rbf * qlin
    pre = (jnp.dot(qi, wea_ref[...], preferred_element_type=jnp.float32)
           + jnp.dot(qj, web_ref[...], preferred_element_type=jnp.float32)
           + jnp.dot(qfilt, wec_ref[...], preferred_element_type=jnp.float32)
           + d * wed_ref[...] + be1_ref[...])
    mtx = (jnp.dot(_silu(pre), we2_ref[...], preferred_element_type=jnp.float32)
           + be2_ref[...])
    z = (jnp.dot(mtx, watt_ref[...], preferred_element_type=jnp.float32)
         + batt_ref[...])
    att = jnp.where(z > 0, z, 2.0 * (jnp.exp(0.5 * z) - 1.0))
    ea = jnp.exp(att)
    rn = r[:, :3] / (d + 1e-5)
    nb = qi.shape[0]
    small = jnp.concatenate(
        [ea, rn, jnp.ones((nb, 1), jnp.float32),
         jnp.zeros((nb, 10), jnp.float32)], axis=1)
    mtx_ref[...] = mtx
    small_ref[...] = small


def edge2_body(mtx_ref, small_ref, sg_ref, wm00_ref, wm01_ref, wm10_ref,
               wm11_ref, p1_ref, p2_ref, p3_ref, p4_ref):
    mtx = mtx_ref[...]
    small = small_ref[...]
    sg = sg_ref[...]
    c0 = small[:, 0:1] * sg[:, 0:1]
    c1 = small[:, 1:2] * sg[:, 1:2]
    m0 = mtx * c0
    m1 = mtx * c1
    pre0 = (jnp.dot(m0, wm00_ref[...], preferred_element_type=jnp.float32)
            + jnp.dot(m1, wm10_ref[...], preferred_element_type=jnp.float32))
    pre1 = (jnp.dot(m0, wm01_ref[...], preferred_element_type=jnp.float32)
            + jnp.dot(m1, wm11_ref[...], preferred_element_type=jnp.float32))
    co0 = jnp.tanh(pre0)
    co1 = jnp.tanh(pre1)
    rn0 = small[:, 2:3]
    rn1 = small[:, 3:4]
    rn2 = small[:, 4:5]
    nb = mtx.shape[0]
    pos = pl.program_id(0) * nb + jax.lax.broadcasted_iota(
        jnp.int32, (nb, 1), 0)
    live = pos < E
    p1_ref[...] = jnp.where(live, jnp.concatenate([m0, m1], axis=1), 0.0)
    p2_ref[...] = jnp.where(
        live, jnp.concatenate([co0 * rn0, co0 * rn1], axis=1), 0.0)
    p3_ref[...] = jnp.where(
        live, jnp.concatenate([co0 * rn2, co1 * rn0], axis=1), 0.0)
    p4_ref[...] = jnp.where(
        live, jnp.concatenate([co1 * rn1, co1 * rn2], axis=1), 0.0)


def sinv_body(s_ref, out_ref):
    s = s_ref[...]
    nb = s.shape[0]
    out_ref[...] = jnp.concatenate(
        [1.0 / s[:, 0:2], jnp.zeros((nb, D - 2), jnp.float32)], axis=1)


def node_body(q_ref, am_ref, a2_ref, a3_ref, a4_ref, s_ref,
              p0_ref, p1_ref, bpn1_ref, wpn2_ref, bpn2_ref,
              wa_ref, g0_ref, g1_ref, cc_ref, bn1_ref, wn2_ref, bn2_ref,
              out_ref):
    q = q_ref[...]
    am = am_ref[...]
    a2 = a2_ref[...]
    a3 = a3_ref[...]
    a4 = a4_ref[...]
    inv = 1.0 / jnp.maximum(s_ref[...][:, 5:6], 1.0)
    u00 = a2[:, :D] * inv
    u01 = a2[:, D:] * inv
    u02 = a3[:, :D] * inv
    u10 = a3[:, D:] * inv
    u11 = a4[:, :D] * inv
    u12 = a4[:, D:] * inv
    cn0 = u00 * u00 + u01 * u01 + u02 * u02
    cn1 = u10 * u10 + u11 * u11 + u12 * u12
    t = _silu(jnp.dot(cn0, p0_ref[...], preferred_element_type=jnp.float32)
              + jnp.dot(cn1, p1_ref[...], preferred_element_type=jnp.float32)
              + bpn1_ref[...])
    q_comb = _silu(jnp.dot(t, wpn2_ref[...], preferred_element_type=jnp.float32)
                   + bpn2_ref[...])
    h = _silu(jnp.dot(q, wa_ref[...], preferred_element_type=jnp.float32)
              + jnp.dot(am[:, :D], g0_ref[...], preferred_element_type=jnp.float32)
              + jnp.dot(am[:, D:], g1_ref[...], preferred_element_type=jnp.float32)
              + jnp.dot(q_comb, cc_ref[...], preferred_element_type=jnp.float32)
              + bn1_ref[...])
    out = _silu(jnp.dot(h, wn2_ref[...], preferred_element_type=jnp.float32)
                + bn2_ref[...])
    out_ref[...] = 2.0 * q + out


def _full(shape):
    return pl.BlockSpec(shape, lambda i: tuple(0 for _ in shape))


def _blk(shape):
    return pl.BlockSpec(shape, lambda i: (i,) + tuple(0 for _ in shape[1:]))


def _tc_edge1(qi, qj, wina, winb, binr, cent, wea, web, wec, wed, be1, we2,
              be2, watt, batt):
    grid = (E_PAD // BE,)
    return pl.pallas_call(
        edge1_body,
        grid=grid,
        in_specs=[
            _blk((BE, D)), _blk((BE, D)),
            _full((D, NRBF)), _full((D, NRBF)), _full((1, NRBF)),
            _full((1, NRBF)),
            _full((D, D)), _full((D, D)), _full((NRBF, D)), _full((1, D)),
            _full((1, D)), _full((D, D)), _full((1, D)),
            _full((D, H)), _full((1, H)),
        ],
        out_specs=[_blk((BE, D)), _blk((BE, 16))],
        out_shape=[
            jax.ShapeDtypeStruct((E_PAD, D), jnp.float32),
            jax.ShapeDtypeStruct((E_PAD, 16), jnp.float32),
        ],
    )(qi, qj, wina, winb, binr, cent, wea, web, wec, wed, be1, we2, be2,
      watt, batt)


def _tc_sinv(s):
    return pl.pallas_call(
        sinv_body,
        grid=(N // BN,),
        in_specs=[_blk((BN, 16))],
        out_specs=_blk((BN, D)),
        out_shape=jax.ShapeDtypeStruct((N, D), jnp.float32),
    )(s)


def _tc_edge2(mtx, small, sg, wm00, wm01, wm10, wm11):
    grid = (E_PAD // BE,)
    return pl.pallas_call(
        edge2_body,
        grid=grid,
        in_specs=[
            _blk((BE, D)), _blk((BE, 16)), _blk((BE, D)),
            _full((D, D)), _full((D, D)), _full((D, D)), _full((D, D)),
        ],
        out_specs=[_blk((BE, C)), _blk((BE, C)), _blk((BE, C)), _blk((BE, C))],
        out_shape=[jax.ShapeDtypeStruct((E_PAD, C), jnp.float32)] * 4,
    )(mtx, small, sg, wm00, wm01, wm10, wm11)


def _tc_node(q, am, a2, a3, a4, s, p0, p1, bpn1, wpn2, bpn2, wa, g0, g1, cc,
             bn1, wn2, bn2):
    grid = (N // BN,)
    return pl.pallas_call(
        node_body,
        grid=grid,
        in_specs=[
            _blk((BN, D)), _blk((BN, C)), _blk((BN, C)), _blk((BN, C)),
            _blk((BN, C)), _blk((BN, 16)),
            _full((D, D)), _full((D, D)), _full((1, D)), _full((D, D)),
            _full((1, D)),
            _full((D, D)), _full((D, D)), _full((D, D)), _full((D, D)),
            _full((1, D)), _full((D, D)), _full((1, D)),
        ],
        out_specs=_blk((BN, D)),
        out_shape=jax.ShapeDtypeStruct((N, D), jnp.float32),
    )(q, am, a2, a3, a4, s, p0, p1, bpn1, wpn2, bpn2, wa, g0, g1, cc, bn1,
      wn2, bn2)


def _gather_q(table, idx2):
    return _make_gather(D)(table, idx2)


def _gather_s(table, idx2):
    return _make_gather(D)(table, idx2)


def _scatter16(jflat, pay):
    return _get_scatter_small()(jflat, pay.reshape(-1)).reshape(N, 16)


_compact_cache = []


def _compact(jflat):
    if not _compact_cache:
        _compact_cache.append(_make_compact())
    return _compact_cache[0](jflat)


def _scatter256(pk, cnt, pay):
    return _make_scatter_wide(C)(pk, cnt, pay)


def kernel(q, mu, pairlist, W_in, b_in, W_e1, b_e1, W_e2, b_e2, W_att, b_att,
           W_mix, W_pn1, b_pn1, W_pn2, b_pn2, W_n1, b_n1, W_n2, b_n2):
    idx_i = pairlist[0].astype(jnp.int32)
    idx_j = pairlist[1].astype(jnp.int32)
    padz = jnp.zeros((E_PAD - E,), jnp.int32)
    iflat = jnp.concatenate([idx_i, padz])
    jflat = jnp.concatenate([idx_j, padz])
    ii2 = iflat.reshape(E_PAD // 128, 128)
    jj2 = jflat.reshape(E_PAD // 128, 128)

    wina, winb = W_in[:D], W_in[D:]
    binr = b_in.reshape(1, NRBF)
    cent = jnp.linspace(0.0, CUTOFF, NRBF).reshape(1, NRBF)
    wea, web = W_e1[:D], W_e1[D:2 * D]
    wec = W_e1[2 * D:2 * D + NRBF]
    wed = W_e1[2 * D + NRBF].reshape(1, D)
    be1 = b_e1.reshape(1, D)
    be2 = b_e2.reshape(1, D)
    batt = b_att.reshape(1, H)
    wm00 = W_mix[0::2, 0::2]
    wm01 = W_mix[0::2, 1::2]
    wm10 = W_mix[1::2, 0::2]
    wm11 = W_mix[1::2, 1::2]
    p0, p1 = W_pn1[0::2], W_pn1[1::2]
    bpn1 = b_pn1.reshape(1, D)
    bpn2 = b_pn2.reshape(1, D)
    wa = W_n1[:D]
    g0 = W_n1[D:D + C][0::2]
    g1 = W_n1[D:D + C][1::2]
    cc = W_n1[D + C:]
    bn1 = b_n1.reshape(1, D)
    bn2 = b_n2.reshape(1, D)

    qi = _gather_q(q, ii2)
    qj = _gather_q(q, jj2)
    mtx, small = _tc_edge1(qi, qj, wina, winb, binr, cent, wea, web, wec,
                           wed, be1, W_e2, be2, W_att, batt)
    s_sum = _scatter16(jflat, small)
    sg = _gather_s(_tc_sinv(s_sum), jj2)
    pay1, pay2, pay3, pay4 = _tc_edge2(mtx, small, sg, wm00, wm01, wm10, wm11)
    pk, cnt = _compact(jflat)
    am = _scatter256(pk, cnt, pay1)
    a2 = _scatter256(pk, cnt, pay2)
    a3 = _scatter256(pk, cnt, pay3)
    a4 = _scatter256(pk, cnt, pay4)
    q_new = _tc_node(q, am, a2, a3, a4, s_sum, p0, p1, bpn1, W_pn2, bpn2,
                     wa, g0, g1, cc, bn1, W_n2, bn2)
    return (q_new, mu)
